# baseline (speedup 1.0000x reference)
import jax
import jax.numpy as jnp
from jax import lax
from jax.experimental import pallas as pl
from jax.experimental.pallas import tpu as pltpu

N = 32768
M = 1024
CNT = 32
DIM = 256
HEADS = 8
HD = DIM // HEADS
GH = 32
GW = 32
ROW_VOX = GW * CNT
F32 = jnp.float32
BF16 = jnp.bfloat16


def _compute_row(t, x_ref, qkvw_ref, selt_ref, mask_ref, q_ref, s_ref):
    xb = x_ref[...]
    qkv = jnp.dot(xb, qkvw_ref[...], preferred_element_type=F32)
    q = jnp.maximum(qkv[:, :DIM], 0.0)
    k = jnp.maximum(qkv[:, DIM:2 * DIM], 0.0)
    v = qkv[:, 2 * DIM:]
    q_ref[t % 2] = q.astype(BF16)

    s_cur = jnp.dot(selt_ref[...], k, preferred_element_type=F32)

    mask = mask_ref[...]
    kv_cur = []
    for m in range(GW):
        lo = max(m - 1, 0) * CNT
        hi = min(m + 2, GW) * CNT
        kvf = lax.dot_general(k[lo:hi], v[lo:hi], (((0,), (0,)), ((), ())),
                              preferred_element_type=F32)
        kv_cur.append((kvf * mask).astype(BF16))
    return kv_cur, s_cur


def _emit_row(t, projw_ref, projb_ref, mask_ref, up_ref, out_ref,
              colsum_ref, q_ref, s_ref, kv_cur, s_cur):
    pp_slot = (t + 1) % 3
    pv_slot = (t + 2) % 3

    qe = q_ref[(t + 1) % 2]
    s_p = s_ref[pp_slot] + s_ref[pv_slot]
    if s_cur is not None:
        s_p = s_p + s_cur
    srows = jnp.dot(up_ref[...], s_p, preferred_element_type=F32)
    zden = jnp.dot(qe.astype(F32) * srows, mask_ref[...],
                   preferred_element_type=F32) + 1e-6

    ys = []
    for m in range(GW):
        kvp = colsum_ref[pp_slot, m] + colsum_ref[pv_slot, m]
        if kv_cur is not None:
            kvp = kvp + kv_cur[m]
        qm = qe[m * CNT:(m + 1) * CNT]
        ys.append(jnp.dot(qm, kvp, preferred_element_type=F32))
    y = jnp.concatenate(ys, axis=0) / zden
    out_ref[...] = (jnp.dot(y, projw_ref[...], preferred_element_type=F32)
                    + projb_ref[...])


def _store_row(t, colsum_ref, s_ref, kv_cur, s_cur):
    s_ref[t % 3] = s_cur
    for m in range(GW):
        colsum_ref[t % 3, m] = kv_cur[m]


def _fused_body(x_ref, qkvw_ref, projw_ref, projb_ref, mask_ref, selt_ref,
                up_ref, out_ref, colsum_ref, q_ref, s_ref):
    t = pl.program_id(0)

    @pl.when(t == 0)
    def _first():
        colsum_ref[2] = jnp.zeros((GW, DIM, DIM), BF16)
        s_ref[2] = jnp.zeros((GW, DIM), F32)
        kv_cur, s_cur = _compute_row(t, x_ref, qkvw_ref, selt_ref, mask_ref,
                                     q_ref, s_ref)
        _store_row(t, colsum_ref, s_ref, kv_cur, s_cur)

    @pl.when((t >= 1) & (t < GH))
    def _middle():
        kv_cur, s_cur = _compute_row(t, x_ref, qkvw_ref, selt_ref, mask_ref,
                                     q_ref, s_ref)
        _emit_row(t, projw_ref, projb_ref, mask_ref, up_ref, out_ref,
                  colsum_ref, q_ref, s_ref, kv_cur, s_cur)
        _store_row(t, colsum_ref, s_ref, kv_cur, s_cur)

    @pl.when(t == GH)
    def _drain():
        _emit_row(t, projw_ref, projb_ref, mask_ref, up_ref, out_ref,
                  colsum_ref, q_ref, s_ref, None, None)


def kernel(x, qkv_w, proj_w, proj_b, offsets, counts, batch_win_inds,
           batch_win_coords):
    del offsets, counts, batch_win_inds, batch_win_coords

    rg = lax.broadcasted_iota(jnp.int32, (DIM, DIM), 0) // HD
    cg = lax.broadcasted_iota(jnp.int32, (DIM, DIM), 1) // HD
    mask = (rg == cg).astype(F32)
    mw = lax.broadcasted_iota(jnp.int32, (GW, ROW_VOX), 0)
    rw = lax.broadcasted_iota(jnp.int32, (GW, ROW_VOX), 1) // CNT
    selt = (jnp.abs(mw - rw) <= 1).astype(F32)
    ri = lax.broadcasted_iota(jnp.int32, (ROW_VOX, GW), 0) // CNT
    ci = lax.broadcasted_iota(jnp.int32, (ROW_VOX, GW), 1)
    up = (ri == ci).astype(F32)

    out = pl.pallas_call(
        _fused_body,
        grid=(GH + 1,),
        in_specs=[
            pl.BlockSpec((ROW_VOX, DIM),
                         lambda t: (jnp.minimum(t, GH - 1), 0)),
            pl.BlockSpec((DIM, 3 * DIM), lambda t: (0, 0)),
            pl.BlockSpec((DIM, DIM), lambda t: (0, 0)),
            pl.BlockSpec((1, DIM), lambda t: (0, 0)),
            pl.BlockSpec((DIM, DIM), lambda t: (0, 0)),
            pl.BlockSpec((GW, ROW_VOX), lambda t: (0, 0)),
            pl.BlockSpec((ROW_VOX, GW), lambda t: (0, 0)),
        ],
        out_specs=pl.BlockSpec((ROW_VOX, DIM),
                               lambda t: (jnp.maximum(t - 1, 0), 0)),
        out_shape=jax.ShapeDtypeStruct((N, DIM), F32),
        scratch_shapes=[
            pltpu.VMEM((3, GW, DIM, DIM), BF16),
            pltpu.VMEM((2, ROW_VOX, DIM), BF16),
            pltpu.VMEM((3, GW, DIM), F32),
        ],
    )(x, qkv_w, proj_w, proj_b.reshape(1, DIM), mask, selt, up)
    return out


# speedup vs baseline: 1.3014x; 1.3014x over previous
import jax
import jax.numpy as jnp
from jax import lax
from jax.experimental import pallas as pl
from jax.experimental.pallas import tpu as pltpu

N = 32768
M = 1024
CNT = 32
DIM = 256
HEADS = 8
HD = DIM // HEADS
GH = 32
GW = 32
ROW_VOX = GW * CNT
PAIR_VOX = 2 * ROW_VOX
STEPS = GH // 2 + 1
RING = 5
F32 = jnp.float32
BF16 = jnp.bfloat16


def _fused_body(x_ref, qkvw_ref, projw_ref, projb_ref, mask_ref, selt_ref,
                up_ref, out_ref, colsum_ref, q_ref, s_ref):
    t = pl.program_id(0)

    @pl.when(t == 0)
    def _zero_top():
        colsum_ref[4] = jnp.zeros((GW, DIM, DIM), BF16)
        s_ref[4] = jnp.zeros((GW, DIM), F32)

    @pl.when(t == STEPS - 1)
    def _zero_bottom():
        colsum_ref[2] = jnp.zeros((GW, DIM, DIM), BF16)
        s_ref[2] = jnp.zeros((GW, DIM), F32)

    @pl.when(t < STEPS - 1)
    def _compute():
        xb = x_ref[...]
        qkv = jnp.dot(xb, qkvw_ref[...], preferred_element_type=F32)
        q = jnp.maximum(qkv[:, :DIM], 0.0)
        k = jnp.maximum(qkv[:, DIM:2 * DIM], 0.0)
        v = qkv[:, 2 * DIM:]
        q_ref[t % 2] = q.astype(BF16)

        mask = mask_ref[...]
        selt = selt_ref[...]
        for half in range(2):
            row = 2 * t + half
            kh = k[half * ROW_VOX:(half + 1) * ROW_VOX]
            vh = v[half * ROW_VOX:(half + 1) * ROW_VOX]
            s_ref[row % RING] = jnp.dot(selt, kh, preferred_element_type=F32)
            for m in range(GW):
                lo = max(m - 1, 0) * CNT
                hi = min(m + 2, GW) * CNT
                kvf = lax.dot_general(kh[lo:hi], vh[lo:hi],
                                      (((0,), (0,)), ((), ())),
                                      preferred_element_type=F32)
                colsum_ref[row % RING, m] = (kvf * mask).astype(BF16)

    @pl.when(t >= 1)
    def _emit():
        qpair = q_ref[(t + 1) % 2]
        outs = []
        for half in range(2):
            r = 2 * t - 2 + half
            prev_slot = (r + RING - 1) % RING
            cur_slot = r % RING
            next_slot = (r + 1) % RING

            qe = qpair[half * ROW_VOX:(half + 1) * ROW_VOX]
            s_p = (s_ref[prev_slot] + s_ref[cur_slot]
                   + s_ref[next_slot])
            srows = jnp.dot(up_ref[...], s_p, preferred_element_type=F32)
            zden = jnp.dot(qe.astype(F32) * srows, mask_ref[...],
                           preferred_element_type=F32) + 1e-6

            ys = []
            for m in range(GW):
                kvp = (colsum_ref[prev_slot, m] + colsum_ref[cur_slot, m]
                       + colsum_ref[next_slot, m])
                qm = qe[m * CNT:(m + 1) * CNT]
                ys.append(jnp.dot(qm, kvp, preferred_element_type=F32))
            y = jnp.concatenate(ys, axis=0) / zden
            outs.append(jnp.dot(y, projw_ref[...],
                                preferred_element_type=F32) + projb_ref[...])
        out_ref[...] = jnp.concatenate(outs, axis=0)


def kernel(x, qkv_w, proj_w, proj_b, offsets, counts, batch_win_inds,
           batch_win_coords):
    del offsets, counts, batch_win_inds, batch_win_coords

    rg = lax.broadcasted_iota(jnp.int32, (DIM, DIM), 0) // HD
    cg = lax.broadcasted_iota(jnp.int32, (DIM, DIM), 1) // HD
    mask = (rg == cg).astype(F32)
    mw = lax.broadcasted_iota(jnp.int32, (GW, ROW_VOX), 0)
    rw = lax.broadcasted_iota(jnp.int32, (GW, ROW_VOX), 1) // CNT
    selt = (jnp.abs(mw - rw) <= 1).astype(F32)
    ri = lax.broadcasted_iota(jnp.int32, (ROW_VOX, GW), 0) // CNT
    ci = lax.broadcasted_iota(jnp.int32, (ROW_VOX, GW), 1)
    up = (ri == ci).astype(F32)

    out = pl.pallas_call(
        _fused_body,
        grid=(STEPS,),
        in_specs=[
            pl.BlockSpec((PAIR_VOX, DIM),
                         lambda t: (jnp.minimum(t, STEPS - 2), 0)),
            pl.BlockSpec((DIM, 3 * DIM), lambda t: (0, 0)),
            pl.BlockSpec((DIM, DIM), lambda t: (0, 0)),
            pl.BlockSpec((1, DIM), lambda t: (0, 0)),
            pl.BlockSpec((DIM, DIM), lambda t: (0, 0)),
            pl.BlockSpec((GW, ROW_VOX), lambda t: (0, 0)),
            pl.BlockSpec((ROW_VOX, GW), lambda t: (0, 0)),
        ],
        out_specs=pl.BlockSpec((PAIR_VOX, DIM),
                               lambda t: (jnp.maximum(t - 1, 0), 0)),
        out_shape=jax.ShapeDtypeStruct((N, DIM), F32),
        scratch_shapes=[
            pltpu.VMEM((RING, GW, DIM, DIM), BF16),
            pltpu.VMEM((2, PAIR_VOX, DIM), BF16),
            pltpu.VMEM((RING, GW, DIM), F32),
        ],
    )(x, qkv_w, proj_w, proj_b.reshape(1, DIM), mask, selt, up)
    return out
